# baseline (speedup 1.0000x reference)
import functools

import jax
import jax.numpy as jnp
from jax import lax
from jax.experimental import pallas as pl
from jax.experimental.pallas import tpu as pltpu
from jax.experimental.pallas import tpu_sc as plsc

N = 320000
D = 128
H = 32
B = 10000

KR = 2000
NBA = N // KR


def _scores_body(x_ref, w1_ref, b1_ref, w2_ref, b2_ref, s_ref, m_ref):
    h = jnp.tanh(
        jnp.dot(x_ref[...], w1_ref[...], preferred_element_type=jnp.float32)
        + b1_ref[...]
    )
    val = jnp.sum(h * w2_ref[...], axis=1) + b2_ref[0, 0]
    s_ref[0] = val.reshape(KR // 8, 8).T

    @pl.when(pl.program_id(0) == 0)
    def _():
        m_ref[0, 0] = -jnp.inf

    m_ref[0, 0] = jnp.maximum(m_ref[0, 0], jnp.max(val))


def _scores(x, W1, b1r, W2r, b2r):
    return pl.pallas_call(
        _scores_body,
        grid=(NBA,),
        in_specs=[
            pl.BlockSpec((KR, D), lambda i: (i, 0)),
            pl.BlockSpec((D, H), lambda i: (0, 0)),
            pl.BlockSpec((1, H), lambda i: (0, 0)),
            pl.BlockSpec((1, H), lambda i: (0, 0)),
            pl.BlockSpec((1, 1), lambda i: (0, 0), memory_space=pltpu.SMEM),
        ],
        out_specs=[
            pl.BlockSpec((1, 8, KR // 8), lambda i: (i, 0, 0)),
            pl.BlockSpec((1, 1), lambda i: (0, 0), memory_space=pltpu.SMEM),
        ],
        out_shape=[
            jax.ShapeDtypeStruct((NBA, 8, KR // 8), jnp.float32),
            jax.ShapeDtypeStruct((1, 1), jnp.float32),
        ],
    )(x, W1, b1r, W2r, b2r)


NC = 2
NS = 16
L = 16
NW = NC * NS
RPT = N // NW
KB = 80
NBLK = RPT // KB
BPC = 5
CHK = BPC * KB
NCH = NBLK // BPC
BP = 10240
SPT = BP // NS


FB = 16
NCC = D // L


def _sc_pool_body(x_hbm, batch_hbm, s_hbm, m_hbm, out_hbm, den_hbm,
                  acc, xb0, xb1, xb2, sb, bidx, den, mv, fb, fidx,
                  g0, g1, g2):
    cid = lax.axis_index("c")
    sid = lax.axis_index("s")
    wid = cid * NS + sid
    base = wid * RPT
    xbufs = [xb0, xb1, xb2]
    gsems = [g0, g1, g2]

    pltpu.sync_copy(m_hbm, mv)
    mvec = mv[...]
    zvec = jnp.zeros((L,), jnp.float32)
    lanes = lax.iota(jnp.int32, L)
    dumpv = jnp.full((L,), BP - 1, jnp.int32)

    def zden(i, carry):
        den[pl.ds(i * L, L)] = zvec
        return carry

    lax.fori_loop(0, BP // L, zden, 0)

    def zrow(r, carry):
        wr = xb0.at[r]
        for c in range(NCC):
            wr[pl.ds(c * L, L)] = zvec
        return carry

    lax.fori_loop(0, KB, zrow, 0)
    for k in range(SPT // KB):
        pltpu.sync_copy(xb0, acc.at[pl.ds(sid * SPT + k * KB, KB)])
    plsc.subcore_barrier()


    def flush_fb(st):
        accv, cur, slot, fv = st
        fidx[...] = fv
        pltpu.sync_copy(fb, acc.at[fidx], add=True)
        return (accv, cur, jnp.int32(0), dumpv)

    def close_seg(st):
        accv, cur, slot, fv = st
        fr = fb.at[slot]
        for c in range(NCC):
            fr[pl.ds(c * L, L)] = accv[c]
        fv = jnp.where(lanes == slot, cur, fv)
        slot = slot + 1
        st = (tuple(zvec for _ in range(NCC)), cur, slot, fv)
        return lax.cond(slot >= FB, flush_fb, lambda t: t, st)

    def add_row(accv, buf, r, e):
        xr = buf.at[r]
        return tuple(
            accv[c] + e * xr[pl.ds(c * L, L)] for c in range(NCC)
        )

    def chunk(ci, st):
        pltpu.sync_copy(
            batch_hbm.at[pl.ds(base + ci * CHK, CHK)], bidx.at[pl.ds(0, CHK)]
        )
        pltpu.sync_copy(s_hbm.at[pl.ds(base + ci * CHK, CHK)], sb.at[pl.ds(0, CHK)])

        def expbody(i, c2):
            sb[pl.ds(i * L, L)] = jnp.exp(sb[pl.ds(i * L, L)] - mvec)
            return c2

        lax.fori_loop(0, CHK // L, expbody, 0)
        cb = base + ci * CHK

        def gather_start(j):
            return pltpu.async_copy(
                x_hbm.at[pl.ds(cb + j * KB, KB)], xbufs[j % 3], gsems[j % 3]
            )

        gds = {j: gather_start(j) for j in range(min(2, BPC))}
        for j in range(BPC):
            if j + 2 < BPC:
                gds[j + 2] = gather_start(j + 2)
            gds[j].wait()
            buf = xbufs[j % 3]

            def grp(g, st2, _j=j, _buf=buf):
                rowc = _j * KB + g * L
                evec = sb[pl.ds(rowc, L)]
                bvec = bidx[pl.ds(rowc, L)]
                plsc.addupdate_scatter(den, [bvec], evec)
                b0 = bvec[0]
                b15 = bvec[L - 1]

                def whole_group(st3):
                    accv, cur, slot, fv = st3
                    for k in range(L):
                        accv = add_row(accv, _buf, g * L + k, evec[k])
                    return (accv, cur, slot, fv)

                def fast_new(st3):
                    accv, cur, slot, fv = close_seg(st3)
                    return whole_group((accv, b0, slot, fv))

                def slow(st3):
                    def slowrow(k, st4):
                        kk = rowc + k
                        e = sb[pl.ds(kk, L)][0]
                        bk = bidx[pl.ds(kk, L)][0]

                        def openrow(st5):
                            accv, cur, slot, fv = close_seg(st5)
                            accv = add_row(accv, _buf, g * L + k, e)
                            return (accv, bk, slot, fv)

                        def addrow(st5):
                            accv, cur, slot, fv = st5
                            accv = add_row(accv, _buf, g * L + k, e)
                            return (accv, cur, slot, fv)

                        return lax.cond(bk != st4[1], openrow, addrow, st4)

                    return lax.fori_loop(0, L, slowrow, st3)

                def uniform_case(st3):
                    return lax.cond(b0 == st3[1], whole_group, fast_new, st3)

                return lax.cond(b0 == b15, uniform_case, slow, st2)

            st = lax.fori_loop(0, KB // L, grp, st)
        return st

    cur0 = jnp.int32(BP - 1)
    st0 = (tuple(zvec for _ in range(NCC)), cur0, jnp.int32(0), dumpv)
    st = lax.fori_loop(0, NCH, chunk, st0)
    st = close_seg(st)
    flush_fb(st)
    plsc.subcore_barrier()

    pltpu.sync_copy(den, den_hbm.at[wid])
    pltpu.sync_copy(
        acc.at[pl.ds(sid * SPT, SPT)],
        out_hbm.at[pl.ds(cid * BP + sid * SPT, SPT)],
    )


_sc_pool = functools.partial(
    pl.kernel,
    out_type=[
        jax.ShapeDtypeStruct((NC * BP, D), jnp.float32),
        jax.ShapeDtypeStruct((NW, BP), jnp.float32),
    ],
    mesh=plsc.VectorSubcoreMesh(core_axis_name="c", subcore_axis_name="s"),
    compiler_params=pltpu.CompilerParams(
        needs_layout_passes=False, use_tc_tiling_on_sc=False
    ),
    scratch_types=[
        pltpu.VMEM_SHARED((BP, D), jnp.float32),
        pltpu.VMEM((KB, D), jnp.float32),
        pltpu.VMEM((KB, D), jnp.float32),
        pltpu.VMEM((KB, D), jnp.float32),
        pltpu.VMEM((CHK + L,), jnp.float32),
        pltpu.VMEM((CHK + L,), jnp.int32),
        pltpu.VMEM((BP,), jnp.float32),
        pltpu.VMEM((L,), jnp.float32),
        pltpu.VMEM((FB, D), jnp.float32),
        pltpu.VMEM((L,), jnp.int32),
        pltpu.SemaphoreType.DMA,
        pltpu.SemaphoreType.DMA,
        pltpu.SemaphoreType.DMA,
    ],
)(_sc_pool_body)


RC = 1024
NBC = (B + RC - 1) // RC


def _combine_body(p0_ref, p1_ref, d_ref, o_ref):
    a = p0_ref[0] + p1_ref[0]
    den = jnp.sum(d_ref[...], axis=0).reshape(RC, 1)
    o_ref[...] = a / (den + 1e-8)


def _combine(p, dp):
    return pl.pallas_call(
        _combine_body,
        grid=(NBC,),
        in_specs=[
            pl.BlockSpec((1, RC, D), lambda i: (0, i, 0)),
            pl.BlockSpec((1, RC, D), lambda i: (1, i, 0)),
            pl.BlockSpec((NW, RC), lambda i: (0, i)),
        ],
        out_specs=pl.BlockSpec((RC, D), lambda i: (i, 0)),
        out_shape=jax.ShapeDtypeStruct((B, D), jnp.float32),
    )(p, p, dp)


def kernel(x, batch, W1, b1, W2, b2):
    b1r = b1.reshape(1, H)
    W2r = W2.reshape(1, H)
    b2r = b2.reshape(1, 1)
    s3, m = _scores(x, W1, b1r, W2r, b2r)
    s = s3.transpose(0, 2, 1).reshape(N)
    mv = jnp.broadcast_to(m.reshape(()), (L,))
    batch1d = batch.astype(jnp.int32).reshape(N)
    partials, den_parts = _sc_pool(x, batch1d, s, mv)
    return _combine(partials.reshape(NC, BP, D), den_parts)


# speedup vs baseline: 1.5547x; 1.5547x over previous
import functools

import jax
import jax.numpy as jnp
from jax import lax
from jax.experimental import pallas as pl
from jax.experimental.pallas import tpu as pltpu
from jax.experimental.pallas import tpu_sc as plsc

N = 320000
D = 128
H = 32
B = 10000

KR = 4000
NBA = N // KR


def _scores_body(x_ref, w1_ref, b1_ref, w2_ref, b2_ref, s_ref, m_ref):
    h = jnp.tanh(
        jnp.dot(x_ref[...], w1_ref[...], preferred_element_type=jnp.float32)
        + b1_ref[...]
    )
    val = jnp.sum(h * w2_ref[...], axis=1) + b2_ref[0, 0]
    s_ref[0] = val.reshape(KR // 8, 8).T

    @pl.when(pl.program_id(0) == 0)
    def _():
        m_ref[0, 0] = -jnp.inf

    m_ref[0, 0] = jnp.maximum(m_ref[0, 0], jnp.max(val))


def _scores(x, W1, b1r, W2r, b2r):
    return pl.pallas_call(
        _scores_body,
        grid=(NBA,),
        in_specs=[
            pl.BlockSpec((KR, D), lambda i: (i, 0)),
            pl.BlockSpec((D, H), lambda i: (0, 0)),
            pl.BlockSpec((1, H), lambda i: (0, 0)),
            pl.BlockSpec((1, H), lambda i: (0, 0)),
            pl.BlockSpec((1, 1), lambda i: (0, 0), memory_space=pltpu.SMEM),
        ],
        out_specs=[
            pl.BlockSpec((1, 8, KR // 8), lambda i: (i, 0, 0)),
            pl.BlockSpec((1, 1), lambda i: (0, 0), memory_space=pltpu.SMEM),
        ],
        out_shape=[
            jax.ShapeDtypeStruct((NBA, 8, KR // 8), jnp.float32),
            jax.ShapeDtypeStruct((1, 1), jnp.float32),
        ],
    )(x, W1, b1r, W2r, b2r)


NC = 2
NS = 16
L = 16
NW = NC * NS
RPT = N // NW
KB = 80
NBLK = RPT // KB
BPC = 25
CHK = BPC * KB
NCH = NBLK // BPC
BP = 10240
SPT = BP // NS


def _sc_pool_body(x_hbm, batch_hbm, s_hbm, m_hbm, out_hbm, den_hbm,
                  acc, xb0, xb1, xb2, sb, bidx, den, mv,
                  g0, g1, g2, s0, s1, s2):
    cid = lax.axis_index("c")
    sid = lax.axis_index("s")
    wid = cid * NS + sid
    base = wid * RPT
    xbufs = [xb0, xb1, xb2]
    gsems = [g0, g1, g2]
    ssems = [s0, s1, s2]

    pltpu.sync_copy(m_hbm, mv)
    mvec = mv[...]
    zvec = jnp.zeros((L,), jnp.float32)

    def zden(i, carry):
        den[pl.ds(i * L, L)] = zvec
        return carry

    lax.fori_loop(0, BP // L, zden, 0)

    def zrow(r, carry):
        wr = xb0.at[r]
        for c in range(D // L):
            wr[pl.ds(c * L, L)] = zvec
        return carry

    lax.fori_loop(0, KB, zrow, 0)
    for k in range(SPT // KB):
        pltpu.sync_copy(xb0, acc.at[pl.ds(sid * SPT + k * KB, KB)])
    plsc.subcore_barrier()

    def chunk(ci, carry):
        pltpu.sync_copy(batch_hbm.at[wid * NCH + ci], bidx)
        pltpu.sync_copy(s_hbm.at[pl.ds(base + ci * CHK, CHK)], sb)

        def expbody(i, c2):
            sb[pl.ds(i * L, L)] = jnp.exp(sb[pl.ds(i * L, L)] - mvec)
            return c2

        lax.fori_loop(0, CHK // L, expbody, 0)
        cb = base + ci * CHK

        def gather_start(j):
            return pltpu.async_copy(
                x_hbm.at[pl.ds(cb + j * KB, KB)], xbufs[j % 3], gsems[j % 3]
            )

        gds = {j: gather_start(j) for j in range(min(3, BPC))}
        sds = {}
        for j in range(BPC):
            b = j % 3
            if 3 <= j + 2 < BPC:
                sds[j - 1].wait()
                gds[j + 2] = gather_start(j + 2)
            gds[j].wait()
            buf = xbufs[b]
            bj = bidx.at[j]

            def grp(g, c2, _j=j, _buf=buf, _bj=bj):
                evec = sb[pl.ds(_j * KB + g * L, L)]
                bvec = _bj[pl.ds(g * L, L)]
                plsc.addupdate_scatter(den, [bvec], evec)
                for k in range(L):
                    e = evec[k]
                    r = g * L + k
                    xr = _buf.at[r]
                    for c in range(D // L):
                        xr[pl.ds(c * L, L)] = e * xr[pl.ds(c * L, L)]
                return c2

            lax.fori_loop(0, KB // L, grp, 0)
            sds[j] = pltpu.async_copy(buf, acc.at[bj], ssems[b], add=True)
        for j in range(max(0, BPC - 3), BPC):
            sds[j].wait()
        return carry

    lax.fori_loop(0, NCH, chunk, 0)
    plsc.subcore_barrier()

    pltpu.sync_copy(den, den_hbm.at[wid])
    pltpu.sync_copy(
        acc.at[pl.ds(sid * SPT, SPT)],
        out_hbm.at[pl.ds(cid * BP + sid * SPT, SPT)],
    )


_sc_pool = functools.partial(
    pl.kernel,
    out_type=[
        jax.ShapeDtypeStruct((NC * BP, D), jnp.float32),
        jax.ShapeDtypeStruct((NW, BP), jnp.float32),
    ],
    mesh=plsc.VectorSubcoreMesh(core_axis_name="c", subcore_axis_name="s"),
    compiler_params=pltpu.CompilerParams(
        needs_layout_passes=False, use_tc_tiling_on_sc=False
    ),
    scratch_types=[
        pltpu.VMEM_SHARED((BP, D), jnp.float32),
        pltpu.VMEM((KB, D), jnp.float32),
        pltpu.VMEM((KB, D), jnp.float32),
        pltpu.VMEM((KB, D), jnp.float32),
        pltpu.VMEM((CHK,), jnp.float32),
        pltpu.VMEM((BPC, KB), jnp.int32),
        pltpu.VMEM((BP,), jnp.float32),
        pltpu.VMEM((L,), jnp.float32),
        pltpu.SemaphoreType.DMA,
        pltpu.SemaphoreType.DMA,
        pltpu.SemaphoreType.DMA,
        pltpu.SemaphoreType.DMA,
        pltpu.SemaphoreType.DMA,
        pltpu.SemaphoreType.DMA,
    ],
)(_sc_pool_body)


RC = 1024
NBC = (B + RC - 1) // RC


def _combine_body(p0_ref, p1_ref, d_ref, o_ref):
    a = p0_ref[0] + p1_ref[0]
    den = jnp.sum(d_ref[...], axis=0).reshape(RC, 1)
    o_ref[...] = a / (den + 1e-8)


def _combine(p, dp):
    return pl.pallas_call(
        _combine_body,
        grid=(NBC,),
        in_specs=[
            pl.BlockSpec((1, RC, D), lambda i: (0, i, 0)),
            pl.BlockSpec((1, RC, D), lambda i: (1, i, 0)),
            pl.BlockSpec((NW, RC), lambda i: (0, i)),
        ],
        out_specs=pl.BlockSpec((RC, D), lambda i: (i, 0)),
        out_shape=jax.ShapeDtypeStruct((B, D), jnp.float32),
    )(p, p, dp)


def kernel(x, batch, W1, b1, W2, b2):
    b1r = b1.reshape(1, H)
    W2r = W2.reshape(1, H)
    b2r = b2.reshape(1, 1)
    s3, m = _scores(x, W1, b1r, W2r, b2r)
    s = s3.transpose(0, 2, 1).reshape(N)
    mv = jnp.broadcast_to(m.reshape(()), (L,))
    batch3d = batch.astype(jnp.int32).reshape(NW * NCH, BPC, KB)
    partials, den_parts = _sc_pool(x, batch3d, s, mv)
    return _combine(partials.reshape(NC, BP, D), den_parts)


# speedup vs baseline: 1.7026x; 1.0952x over previous
import functools

import jax
import jax.numpy as jnp
from jax import lax
from jax.experimental import pallas as pl
from jax.experimental.pallas import tpu as pltpu
from jax.experimental.pallas import tpu_sc as plsc

N = 320000
D = 128
H = 32
B = 10000

KR = 8000
NBA = N // KR


def _scores_body(x_ref, w1_ref, b1_ref, w2_ref, b2_ref, s_ref, m_ref):
    h = jnp.tanh(
        jnp.dot(x_ref[...], w1_ref[...], preferred_element_type=jnp.float32)
        + b1_ref[...]
    )
    val = jnp.sum(h * w2_ref[...], axis=1) + b2_ref[0, 0]
    s_ref[0] = val.reshape(KR // 8, 8).T

    @pl.when(pl.program_id(0) == 0)
    def _():
        m_ref[0, 0] = -jnp.inf

    m_ref[0, 0] = jnp.maximum(m_ref[0, 0], jnp.max(val))


def _scores(x, W1, b1r, W2r, b2r):
    return pl.pallas_call(
        _scores_body,
        grid=(NBA,),
        in_specs=[
            pl.BlockSpec((KR, D), lambda i: (i, 0)),
            pl.BlockSpec((D, H), lambda i: (0, 0)),
            pl.BlockSpec((1, H), lambda i: (0, 0)),
            pl.BlockSpec((1, H), lambda i: (0, 0)),
            pl.BlockSpec((1, 1), lambda i: (0, 0), memory_space=pltpu.SMEM),
        ],
        out_specs=[
            pl.BlockSpec((1, 8, KR // 8), lambda i: (i, 0, 0)),
            pl.BlockSpec((1, 1), lambda i: (0, 0), memory_space=pltpu.SMEM),
        ],
        out_shape=[
            jax.ShapeDtypeStruct((NBA, 8, KR // 8), jnp.float32),
            jax.ShapeDtypeStruct((1, 1), jnp.float32),
        ],
    )(x, W1, b1r, W2r, b2r)


NC = 2
NS = 16
L = 16
NW = NC * NS
RPT = N // NW
KB = 80
NBLK = RPT // KB
BPC = 25
CHK = BPC * KB
NCH = NBLK // BPC
BP = 10240
SPT = BP // NS


def _sc_pool_body(x_hbm, batch_hbm, s_hbm, m_hbm, out_hbm, den_hbm,
                  acc, xb0, xb1, xb2, sb, bidx, den, mv,
                  g0, g1, g2, s0, s1, s2):
    cid = lax.axis_index("c")
    sid = lax.axis_index("s")
    wid = cid * NS + sid
    base = wid * RPT
    xbufs = [xb0, xb1, xb2]
    gsems = [g0, g1, g2]
    ssems = [s0, s1, s2]

    pltpu.sync_copy(m_hbm, mv)
    mvec = mv[...]
    zvec = jnp.zeros((L,), jnp.float32)

    def zden(i, carry):
        den[pl.ds(i * L, L)] = zvec
        return carry

    lax.fori_loop(0, BP // L, zden, 0)

    def zrow(r, carry):
        wr = xb0.at[r]
        for c in range(D // L):
            wr[pl.ds(c * L, L)] = zvec
        return carry

    lax.fori_loop(0, KB, zrow, 0)
    for k in range(SPT // KB):
        pltpu.sync_copy(xb0, acc.at[pl.ds(sid * SPT + k * KB, KB)])
    plsc.subcore_barrier()

    def chunk(ci, carry):
        pltpu.sync_copy(batch_hbm.at[wid * NCH + ci], bidx)
        pltpu.sync_copy(s_hbm.at[pl.ds(base + ci * CHK, CHK)], sb)

        def expbody(i, c2):
            sb[pl.ds(i * L, L)] = jnp.exp(sb[pl.ds(i * L, L)] - mvec)
            return c2

        lax.fori_loop(0, CHK // L, expbody, 0)
        cb = base + ci * CHK

        def gather_start(j):
            return pltpu.async_copy(
                x_hbm.at[pl.ds(cb + j * KB, KB)], xbufs[j % 3], gsems[j % 3]
            )

        gds = {j: gather_start(j) for j in range(min(3, BPC))}
        sds = {}
        for j in range(BPC):
            b = j % 3
            if 3 <= j + 2 < BPC:
                sds[j - 1].wait()
                gds[j + 2] = gather_start(j + 2)
            gds[j].wait()
            buf = xbufs[b]
            bj = bidx.at[j]

            def grp(g, c2, _j=j, _buf=buf, _bj=bj):
                evec = sb[pl.ds(_j * KB + g * L, L)]
                bvec = _bj[pl.ds(g * L, L)]
                plsc.addupdate_scatter(den, [bvec], evec)
                for k in range(L):
                    e = evec[k]
                    r = g * L + k
                    xr = _buf.at[r]
                    for c in range(D // L):
                        xr[pl.ds(c * L, L)] = e * xr[pl.ds(c * L, L)]
                return c2

            lax.fori_loop(0, KB // L, grp, 0)
            sds[j] = pltpu.async_copy(buf, acc.at[bj], ssems[b], add=True)
        for j in range(max(0, BPC - 3), BPC):
            sds[j].wait()
        return carry

    lax.fori_loop(0, NCH, chunk, 0)
    plsc.subcore_barrier()

    pltpu.sync_copy(den, den_hbm.at[wid])
    pltpu.sync_copy(
        acc.at[pl.ds(sid * SPT, SPT)],
        out_hbm.at[pl.ds(cid * BP + sid * SPT, SPT)],
    )


_sc_pool = functools.partial(
    pl.kernel,
    out_type=[
        jax.ShapeDtypeStruct((NC * BP, D), jnp.float32),
        jax.ShapeDtypeStruct((NW, BP), jnp.float32),
    ],
    mesh=plsc.VectorSubcoreMesh(core_axis_name="c", subcore_axis_name="s"),
    compiler_params=pltpu.CompilerParams(
        needs_layout_passes=False, use_tc_tiling_on_sc=False
    ),
    scratch_types=[
        pltpu.VMEM_SHARED((BP, D), jnp.float32),
        pltpu.VMEM((KB, D), jnp.float32),
        pltpu.VMEM((KB, D), jnp.float32),
        pltpu.VMEM((KB, D), jnp.float32),
        pltpu.VMEM((CHK,), jnp.float32),
        pltpu.VMEM((BPC, KB), jnp.int32),
        pltpu.VMEM((BP,), jnp.float32),
        pltpu.VMEM((L,), jnp.float32),
        pltpu.SemaphoreType.DMA,
        pltpu.SemaphoreType.DMA,
        pltpu.SemaphoreType.DMA,
        pltpu.SemaphoreType.DMA,
        pltpu.SemaphoreType.DMA,
        pltpu.SemaphoreType.DMA,
    ],
)(_sc_pool_body)


RC = 1024
NBC = (B + RC - 1) // RC


def _combine_body(p0_ref, p1_ref, d_ref, o_ref):
    a = p0_ref[0] + p1_ref[0]
    den = jnp.sum(d_ref[...], axis=0).reshape(RC, 1)
    o_ref[...] = a / (den + 1e-8)


def _combine(p, dp):
    return pl.pallas_call(
        _combine_body,
        grid=(NBC,),
        in_specs=[
            pl.BlockSpec((1, RC, D), lambda i: (0, i, 0)),
            pl.BlockSpec((1, RC, D), lambda i: (1, i, 0)),
            pl.BlockSpec((NW, RC), lambda i: (0, i)),
        ],
        out_specs=pl.BlockSpec((RC, D), lambda i: (i, 0)),
        out_shape=jax.ShapeDtypeStruct((B, D), jnp.float32),
    )(p, p, dp)


def kernel(x, batch, W1, b1, W2, b2):
    b1r = b1.reshape(1, H)
    W2r = W2.reshape(1, H)
    b2r = b2.reshape(1, 1)
    s3, m = _scores(x, W1, b1r, W2r, b2r)
    s = s3.transpose(0, 2, 1).reshape(N)
    mv = jnp.broadcast_to(m.reshape(()), (L,))
    batch3d = batch.astype(jnp.int32).reshape(NW * NCH, BPC, KB)
    partials, den_parts = _sc_pool(x, batch3d, s, mv)
    return _combine(partials.reshape(NC, BP, D), den_parts)


# speedup vs baseline: 1.7781x; 1.0443x over previous
import functools

import jax
import jax.numpy as jnp
from jax import lax
from jax.experimental import pallas as pl
from jax.experimental.pallas import tpu as pltpu
from jax.experimental.pallas import tpu_sc as plsc

N = 320000
D = 128
H = 32
B = 10000

KR = 16000
NBA = N // KR


def _scores_body(x_ref, w1_ref, b1_ref, w2_ref, b2_ref, s_ref, m_ref):
    h = jnp.tanh(
        jnp.dot(x_ref[...], w1_ref[...], preferred_element_type=jnp.float32)
        + b1_ref[...]
    )
    val = jnp.sum(h * w2_ref[...], axis=1) + b2_ref[0, 0]
    s_ref[0] = val.reshape(KR // 8, 8).T

    @pl.when(pl.program_id(0) == 0)
    def _():
        m_ref[0, 0] = -jnp.inf

    m_ref[0, 0] = jnp.maximum(m_ref[0, 0], jnp.max(val))


def _scores(x, W1, b1r, W2r, b2r):
    return pl.pallas_call(
        _scores_body,
        grid=(NBA,),
        in_specs=[
            pl.BlockSpec((KR, D), lambda i: (i, 0)),
            pl.BlockSpec((D, H), lambda i: (0, 0)),
            pl.BlockSpec((1, H), lambda i: (0, 0)),
            pl.BlockSpec((1, H), lambda i: (0, 0)),
            pl.BlockSpec((1, 1), lambda i: (0, 0), memory_space=pltpu.SMEM),
        ],
        out_specs=[
            pl.BlockSpec((1, 8, KR // 8), lambda i: (i, 0, 0)),
            pl.BlockSpec((1, 1), lambda i: (0, 0), memory_space=pltpu.SMEM),
        ],
        out_shape=[
            jax.ShapeDtypeStruct((NBA, 8, KR // 8), jnp.float32),
            jax.ShapeDtypeStruct((1, 1), jnp.float32),
        ],
    )(x, W1, b1r, W2r, b2r)


NC = 2
NS = 16
L = 16
NW = NC * NS
RPT = N // NW
KB = 80
NBLK = RPT // KB
BPC = 25
CHK = BPC * KB
NCH = NBLK // BPC
BP = 10240
SPT = BP // NS


def _sc_pool_body(x_hbm, batch_hbm, s_hbm, m_hbm, out_hbm, den_hbm,
                  acc, xb0, xb1, xb2, sb, bidx, den, mv,
                  g0, g1, g2, s0, s1, s2):
    cid = lax.axis_index("c")
    sid = lax.axis_index("s")
    wid = cid * NS + sid
    base = wid * RPT
    xbufs = [xb0, xb1, xb2]
    gsems = [g0, g1, g2]
    ssems = [s0, s1, s2]

    pltpu.sync_copy(m_hbm, mv)
    mvec = mv[...]
    zvec = jnp.zeros((L,), jnp.float32)

    def zden(i, carry):
        den[pl.ds(i * L, L)] = zvec
        return carry

    lax.fori_loop(0, BP // L, zden, 0)

    def zrow(r, carry):
        wr = xb0.at[r]
        for c in range(D // L):
            wr[pl.ds(c * L, L)] = zvec
        return carry

    lax.fori_loop(0, KB, zrow, 0)
    for k in range(SPT // KB):
        pltpu.sync_copy(xb0, acc.at[pl.ds(sid * SPT + k * KB, KB)])
    plsc.subcore_barrier()

    def chunk(ci, carry):
        pltpu.sync_copy(batch_hbm.at[wid * NCH + ci], bidx)
        pltpu.sync_copy(s_hbm.at[pl.ds(base + ci * CHK, CHK)], sb)

        def expbody(i, c2):
            sb[pl.ds(i * L, L)] = jnp.exp(sb[pl.ds(i * L, L)] - mvec)
            return c2

        lax.fori_loop(0, CHK // L, expbody, 0)
        cb = base + ci * CHK

        def gather_start(j):
            return pltpu.async_copy(
                x_hbm.at[pl.ds(cb + j * KB, KB)], xbufs[j % 3], gsems[j % 3]
            )

        gds = {j: gather_start(j) for j in range(min(3, BPC))}
        sds = {}
        for j in range(BPC):
            b = j % 3
            if 3 <= j + 2 < BPC:
                sds[j - 1].wait()
                gds[j + 2] = gather_start(j + 2)
            gds[j].wait()
            buf = xbufs[b]
            bj = bidx.at[j]

            def grp(g, c2, _j=j, _buf=buf, _bj=bj):
                evec = sb[pl.ds(_j * KB + g * L, L)]
                bvec = _bj[pl.ds(g * L, L)]
                plsc.addupdate_scatter(den, [bvec], evec)
                for k in range(L):
                    e = evec[k]
                    r = g * L + k
                    xr = _buf.at[r]
                    for c in range(D // L):
                        xr[pl.ds(c * L, L)] = e * xr[pl.ds(c * L, L)]
                return c2

            lax.fori_loop(0, KB // L, grp, 0)
            sds[j] = pltpu.async_copy(buf, acc.at[bj], ssems[b], add=True)
        for j in range(max(0, BPC - 3), BPC):
            sds[j].wait()
        return carry

    lax.fori_loop(0, NCH, chunk, 0)
    plsc.subcore_barrier()

    pltpu.sync_copy(den, den_hbm.at[wid])
    pltpu.sync_copy(
        acc.at[pl.ds(sid * SPT, SPT)],
        out_hbm.at[pl.ds(cid * BP + sid * SPT, SPT)],
    )


_sc_pool = functools.partial(
    pl.kernel,
    out_type=[
        jax.ShapeDtypeStruct((NC * BP, D), jnp.float32),
        jax.ShapeDtypeStruct((NW, BP), jnp.float32),
    ],
    mesh=plsc.VectorSubcoreMesh(core_axis_name="c", subcore_axis_name="s"),
    compiler_params=pltpu.CompilerParams(
        needs_layout_passes=False, use_tc_tiling_on_sc=False
    ),
    scratch_types=[
        pltpu.VMEM_SHARED((BP, D), jnp.float32),
        pltpu.VMEM((KB, D), jnp.float32),
        pltpu.VMEM((KB, D), jnp.float32),
        pltpu.VMEM((KB, D), jnp.float32),
        pltpu.VMEM((CHK,), jnp.float32),
        pltpu.VMEM((BPC, KB), jnp.int32),
        pltpu.VMEM((BP,), jnp.float32),
        pltpu.VMEM((L,), jnp.float32),
        pltpu.SemaphoreType.DMA,
        pltpu.SemaphoreType.DMA,
        pltpu.SemaphoreType.DMA,
        pltpu.SemaphoreType.DMA,
        pltpu.SemaphoreType.DMA,
        pltpu.SemaphoreType.DMA,
    ],
)(_sc_pool_body)


RC = 1024
NBC = (B + RC - 1) // RC


def _combine_body(p0_ref, p1_ref, d_ref, o_ref):
    a = p0_ref[0] + p1_ref[0]
    den = jnp.sum(d_ref[...], axis=0).reshape(RC, 1)
    o_ref[...] = a / (den + 1e-8)


def _combine(p, dp):
    return pl.pallas_call(
        _combine_body,
        grid=(NBC,),
        in_specs=[
            pl.BlockSpec((1, RC, D), lambda i: (0, i, 0)),
            pl.BlockSpec((1, RC, D), lambda i: (1, i, 0)),
            pl.BlockSpec((NW, RC), lambda i: (0, i)),
        ],
        out_specs=pl.BlockSpec((RC, D), lambda i: (i, 0)),
        out_shape=jax.ShapeDtypeStruct((B, D), jnp.float32),
    )(p, p, dp)


def kernel(x, batch, W1, b1, W2, b2):
    b1r = b1.reshape(1, H)
    W2r = W2.reshape(1, H)
    b2r = b2.reshape(1, 1)
    s3, m = _scores(x, W1, b1r, W2r, b2r)
    s = s3.transpose(0, 2, 1).reshape(N)
    mv = jnp.broadcast_to(m.reshape(()), (L,))
    batch3d = batch.astype(jnp.int32).reshape(NW * NCH, BPC, KB)
    partials, den_parts = _sc_pool(x, batch3d, s, mv)
    return _combine(partials.reshape(NC, BP, D), den_parts)


# speedup vs baseline: 1.8089x; 1.0173x over previous
import functools

import jax
import jax.numpy as jnp
from jax import lax
from jax.experimental import pallas as pl
from jax.experimental.pallas import tpu as pltpu
from jax.experimental.pallas import tpu_sc as plsc

N = 320000
D = 128
H = 32
B = 10000

KR = 32000
NBA = N // KR


def _scores_body(x_ref, w1_ref, b1_ref, w2_ref, b2_ref, s_ref, m_ref):
    h = jnp.tanh(
        jnp.dot(x_ref[...], w1_ref[...], preferred_element_type=jnp.float32)
        + b1_ref[...]
    )
    val = jnp.sum(h * w2_ref[...], axis=1) + b2_ref[0, 0]
    s_ref[0] = val.reshape(KR // 8, 8).T

    @pl.when(pl.program_id(0) == 0)
    def _():
        m_ref[0, 0] = -jnp.inf

    m_ref[0, 0] = jnp.maximum(m_ref[0, 0], jnp.max(val))


def _scores(x, W1, b1r, W2r, b2r):
    return pl.pallas_call(
        _scores_body,
        grid=(NBA,),
        in_specs=[
            pl.BlockSpec((KR, D), lambda i: (i, 0)),
            pl.BlockSpec((D, H), lambda i: (0, 0)),
            pl.BlockSpec((1, H), lambda i: (0, 0)),
            pl.BlockSpec((1, H), lambda i: (0, 0)),
            pl.BlockSpec((1, 1), lambda i: (0, 0), memory_space=pltpu.SMEM),
        ],
        out_specs=[
            pl.BlockSpec((1, 8, KR // 8), lambda i: (i, 0, 0)),
            pl.BlockSpec((1, 1), lambda i: (0, 0), memory_space=pltpu.SMEM),
        ],
        out_shape=[
            jax.ShapeDtypeStruct((NBA, 8, KR // 8), jnp.float32),
            jax.ShapeDtypeStruct((1, 1), jnp.float32),
        ],
    )(x, W1, b1r, W2r, b2r)


NC = 2
NS = 16
L = 16
NW = NC * NS
RPT = N // NW
KB = 80
NBLK = RPT // KB
BPC = 25
CHK = BPC * KB
NCH = NBLK // BPC
BP = 10240
SPT = BP // NS


def _sc_pool_body(x_hbm, batch_hbm, s_hbm, m_hbm, out_hbm, den_hbm,
                  acc, xb0, xb1, xb2, sb, bidx, den, mv,
                  g0, g1, g2, s0, s1, s2):
    cid = lax.axis_index("c")
    sid = lax.axis_index("s")
    wid = cid * NS + sid
    base = wid * RPT
    xbufs = [xb0, xb1, xb2]
    gsems = [g0, g1, g2]
    ssems = [s0, s1, s2]

    pltpu.sync_copy(m_hbm, mv)
    mvec = mv[...]
    zvec = jnp.zeros((L,), jnp.float32)

    def zden(i, carry):
        den[pl.ds(i * L, L)] = zvec
        return carry

    lax.fori_loop(0, BP // L, zden, 0)

    def zrow(r, carry):
        wr = xb0.at[r]
        for c in range(D // L):
            wr[pl.ds(c * L, L)] = zvec
        return carry

    lax.fori_loop(0, KB, zrow, 0)
    for k in range(SPT // KB):
        pltpu.sync_copy(xb0, acc.at[pl.ds(sid * SPT + k * KB, KB)])
    plsc.subcore_barrier()

    def chunk(ci, carry):
        pltpu.sync_copy(batch_hbm.at[wid * NCH + ci], bidx)
        pltpu.sync_copy(s_hbm.at[pl.ds(base + ci * CHK, CHK)], sb)

        def expbody(i, c2):
            sb[pl.ds(i * L, L)] = jnp.exp(sb[pl.ds(i * L, L)] - mvec)
            return c2

        lax.fori_loop(0, CHK // L, expbody, 0)
        cb = base + ci * CHK

        def gather_start(j):
            return pltpu.async_copy(
                x_hbm.at[pl.ds(cb + j * KB, KB)], xbufs[j % 3], gsems[j % 3]
            )

        gds = {j: gather_start(j) for j in range(min(3, BPC))}
        sds = {}
        for j in range(BPC):
            b = j % 3
            if 3 <= j + 2 < BPC:
                sds[j - 1].wait()
                gds[j + 2] = gather_start(j + 2)
            gds[j].wait()
            buf = xbufs[b]
            bj = bidx.at[j]

            def grp(g, c2, _j=j, _buf=buf, _bj=bj):
                evec = sb[pl.ds(_j * KB + g * L, L)]
                bvec = _bj[pl.ds(g * L, L)]
                plsc.addupdate_scatter(den, [bvec], evec)
                for k in range(L):
                    e = evec[k]
                    r = g * L + k
                    xr = _buf.at[r]
                    for c in range(D // L):
                        xr[pl.ds(c * L, L)] = e * xr[pl.ds(c * L, L)]
                return c2

            lax.fori_loop(0, KB // L, grp, 0)
            sds[j] = pltpu.async_copy(buf, acc.at[bj], ssems[b], add=True)
        for j in range(max(0, BPC - 3), BPC):
            sds[j].wait()
        return carry

    lax.fori_loop(0, NCH, chunk, 0)
    plsc.subcore_barrier()

    pltpu.sync_copy(den, den_hbm.at[wid])
    pltpu.sync_copy(
        acc.at[pl.ds(sid * SPT, SPT)],
        out_hbm.at[pl.ds(cid * BP + sid * SPT, SPT)],
    )


_sc_pool = functools.partial(
    pl.kernel,
    out_type=[
        jax.ShapeDtypeStruct((NC * BP, D), jnp.float32),
        jax.ShapeDtypeStruct((NW, BP), jnp.float32),
    ],
    mesh=plsc.VectorSubcoreMesh(core_axis_name="c", subcore_axis_name="s"),
    compiler_params=pltpu.CompilerParams(
        needs_layout_passes=False, use_tc_tiling_on_sc=False
    ),
    scratch_types=[
        pltpu.VMEM_SHARED((BP, D), jnp.float32),
        pltpu.VMEM((KB, D), jnp.float32),
        pltpu.VMEM((KB, D), jnp.float32),
        pltpu.VMEM((KB, D), jnp.float32),
        pltpu.VMEM((CHK,), jnp.float32),
        pltpu.VMEM((BPC, KB), jnp.int32),
        pltpu.VMEM((BP,), jnp.float32),
        pltpu.VMEM((L,), jnp.float32),
        pltpu.SemaphoreType.DMA,
        pltpu.SemaphoreType.DMA,
        pltpu.SemaphoreType.DMA,
        pltpu.SemaphoreType.DMA,
        pltpu.SemaphoreType.DMA,
        pltpu.SemaphoreType.DMA,
    ],
)(_sc_pool_body)


RC = 1024
NBC = (B + RC - 1) // RC


def _combine_body(p0_ref, p1_ref, d_ref, o_ref):
    a = p0_ref[0] + p1_ref[0]
    den = jnp.sum(d_ref[...], axis=0).reshape(RC, 1)
    o_ref[...] = a / (den + 1e-8)


def _combine(p, dp):
    return pl.pallas_call(
        _combine_body,
        grid=(NBC,),
        in_specs=[
            pl.BlockSpec((1, RC, D), lambda i: (0, i, 0)),
            pl.BlockSpec((1, RC, D), lambda i: (1, i, 0)),
            pl.BlockSpec((NW, RC), lambda i: (0, i)),
        ],
        out_specs=pl.BlockSpec((RC, D), lambda i: (i, 0)),
        out_shape=jax.ShapeDtypeStruct((B, D), jnp.float32),
    )(p, p, dp)


def kernel(x, batch, W1, b1, W2, b2):
    b1r = b1.reshape(1, H)
    W2r = W2.reshape(1, H)
    b2r = b2.reshape(1, 1)
    s3, m = _scores(x, W1, b1r, W2r, b2r)
    s = s3.transpose(0, 2, 1).reshape(N)
    mv = jnp.broadcast_to(m.reshape(()), (L,))
    batch3d = batch.astype(jnp.int32).reshape(NW * NCH, BPC, KB)
    partials, den_parts = _sc_pool(x, batch3d, s, mv)
    return _combine(partials.reshape(NC, BP, D), den_parts)
